# baseline (speedup 1.0000x reference)
import functools

import jax
import jax.numpy as jnp
from jax import lax
from jax.experimental import pallas as pl
from jax.experimental.pallas import tpu as pltpu
from jax.experimental.pallas import tpu_sc as plsc

_B = 128
_MARGIN = 1.0
_EPS = 1e-8
_NC, _NS, _L = 2, 16, 16
_NW = _NC * _NS
_APW = _B // _NW
_NV = _B // _L


def _tc_body(embs_ref, lab_col_ref, lab_row_ref, ap_ref, an_ref):
    e = embs_ref[...]
    g = lax.dot_general(e, e, (((1,), (1,)), ((), ())),
                        preferred_element_type=jnp.float32)
    n2c = jnp.sum(e * e, axis=1, keepdims=True)
    riota = lax.broadcasted_iota(jnp.int32, (_B, _B), 0)
    ciota = lax.broadcasted_iota(jnp.int32, (_B, _B), 1)
    diag_n2 = jnp.where(riota == ciota, jnp.broadcast_to(n2c, (_B, _B)), 0.0)
    n2r = lax.dot_general(jnp.ones((_B, _B), jnp.float32), diag_n2,
                          (((1,), (0,)), ((), ())),
                          preferred_element_type=jnp.float32)
    denom = jnp.maximum(jnp.sqrt(jnp.broadcast_to(n2c, (_B, _B)) * n2r), _EPS)
    s = g / denom

    same = jnp.broadcast_to(lab_col_ref[...], (_B, _B)) == \
        jnp.broadcast_to(lab_row_ref[...], (_B, _B))
    gt = ciota > riota
    ap_ref[...] = jnp.where(gt & same, s, 3.0)
    an_ref[...] = jnp.where(gt & (~same), s + _MARGIN, -3.0)


def _sc_body(ap_hbm, an_hbm, out_hbm, ap_v, an_v, pv, posb):
    wid = lax.axis_index("s") * _NC + lax.axis_index("c")
    pltpu.sync_copy(ap_hbm.at[wid], ap_v)
    pltpu.sync_copy(an_hbm.at[wid], an_v)
    accs = (jnp.zeros((_L,), jnp.float32),) * _NV
    for a in range(_APW):
        nn = [an_v[pl.ds(a * _B + v * _L, _L)] for v in range(_NV)]
        npos = jnp.int32(0)
        for v in range(_NV):
            w16 = ap_v[pl.ds(a * _B + v * _L, _L)]
            m = w16 < 2.0
            rank = plsc.cumsum(m.astype(jnp.int32)) - 1
            plsc.store_scatter(posb, [rank + npos], w16, mask=m)
            npos = npos + lax.reduce_max(
                plsc.all_reduce_population_count(m), (0,))

        def jbody(j, accs, nn=nn):
            p = plsc.load_gather(posb, [jnp.full((_L,), 0, jnp.int32) + j])
            return tuple(accs[v] + jnp.maximum(nn[v] - p, 0.0)
                         for v in range(_NV))

        accs = lax.fori_loop(0, npos, jbody, accs)
    tot = accs[0]
    for v in range(1, _NV):
        tot = tot + accs[v]
    pv[...] = tot
    pltpu.sync_copy(pv, out_hbm.at[wid])


def kernel(embs, indices):
    lab = indices.astype(jnp.int32)
    ap, an = pl.pallas_call(
        _tc_body,
        out_shape=(jax.ShapeDtypeStruct((_B, _B), jnp.float32),
                   jax.ShapeDtypeStruct((_B, _B), jnp.float32)),
    )(embs, lab.reshape(_B, 1), lab.reshape(1, _B))

    sc = pl.kernel(
        _sc_body,
        out_type=jax.ShapeDtypeStruct((_NW, _L), jnp.float32),
        mesh=plsc.VectorSubcoreMesh(core_axis_name="c", subcore_axis_name="s",
                                    num_cores=_NC, num_subcores=_NS),
        compiler_params=pltpu.CompilerParams(needs_layout_passes=False, skip_device_barrier=True),
        scratch_types=[
            pltpu.VMEM((_APW * _B,), jnp.float32),
            pltpu.VMEM((_APW * _B,), jnp.float32),
            pltpu.VMEM((_L,), jnp.float32),
            pltpu.VMEM((_B + _L,), jnp.float32),
        ],
    )
    partials = sc(ap.reshape(_NW, _APW * _B), an.reshape(_NW, _APW * _B))
    return jnp.sum(partials)


# speedup vs baseline: 1.0163x; 1.0163x over previous
import functools

import jax
import jax.numpy as jnp
from jax import lax
from jax.experimental import pallas as pl
from jax.experimental.pallas import tpu as pltpu
from jax.experimental.pallas import tpu_sc as plsc

_B = 128
_MARGIN = 1.0
_EPS = 1e-8
_NC, _NS, _L = 2, 16, 16
_NW = _NC * _NS
_APW = _B // _NW
_NV = _B // _L


def _tc_body(embs_ref, lab_col_ref, lab_row_ref, ap_ref, an_ref):
    e = embs_ref[...]
    g = lax.dot_general(e, e, (((1,), (1,)), ((), ())),
                        preferred_element_type=jnp.float32)
    n2c = jnp.sum(e * e, axis=1, keepdims=True)
    riota = lax.broadcasted_iota(jnp.int32, (_B, _B), 0)
    ciota = lax.broadcasted_iota(jnp.int32, (_B, _B), 1)
    diag_n2 = jnp.where(riota == ciota, jnp.broadcast_to(n2c, (_B, _B)), 0.0)
    n2r = lax.dot_general(jnp.ones((_B, _B), jnp.float32), diag_n2,
                          (((1,), (0,)), ((), ())),
                          preferred_element_type=jnp.float32)
    denom = jnp.maximum(jnp.sqrt(jnp.broadcast_to(n2c, (_B, _B)) * n2r), _EPS)
    s = g / denom

    same = jnp.broadcast_to(lab_col_ref[...], (_B, _B)) == \
        jnp.broadcast_to(lab_row_ref[...], (_B, _B))
    gt = ciota > riota
    ap_ref[...] = jnp.where(gt & same, s, 3.0)
    an_ref[...] = jnp.where(gt & (~same), s + _MARGIN, -3.0)


def _sc_body(ap_hbm, an_hbm, out_hbm, ap_v, an_v, pv, posb):
    wid = lax.axis_index("s") * _NC + lax.axis_index("c")
    pltpu.sync_copy(ap_hbm.at[wid], ap_v)
    pltpu.sync_copy(an_hbm.at[wid], an_v)
    def abody(a, accs):
        nn = [an_v[pl.ds(a * _B + v * _L, _L)] for v in range(_NV)]
        npos = jnp.int32(0)
        for v in range(_NV):
            w16 = ap_v[pl.ds(a * _B + v * _L, _L)]
            m = w16 < 2.0
            rank = plsc.cumsum(m.astype(jnp.int32)) - 1
            plsc.store_scatter(posb, [rank + npos], w16, mask=m)
            npos = npos + lax.reduce_max(
                plsc.all_reduce_population_count(m), (0,))

        def jbody(j, accs, nn=nn):
            p = plsc.load_gather(posb, [jnp.full((_L,), 0, jnp.int32) + j])
            return tuple(accs[v] + jnp.maximum(nn[v] - p, 0.0)
                         for v in range(_NV))

        return lax.fori_loop(0, npos, jbody, accs)

    accs = lax.fori_loop(0, _APW, abody,
                         (jnp.zeros((_L,), jnp.float32),) * _NV)
    tot = accs[0]
    for v in range(1, _NV):
        tot = tot + accs[v]
    pv[...] = tot
    pltpu.sync_copy(pv, out_hbm.at[wid])


def kernel(embs, indices):
    lab = indices.astype(jnp.int32)
    ap, an = pl.pallas_call(
        _tc_body,
        out_shape=(jax.ShapeDtypeStruct((_B, _B), jnp.float32),
                   jax.ShapeDtypeStruct((_B, _B), jnp.float32)),
    )(embs, lab.reshape(_B, 1), lab.reshape(1, _B))

    sc = pl.kernel(
        _sc_body,
        out_type=jax.ShapeDtypeStruct((_NW, _L), jnp.float32),
        mesh=plsc.VectorSubcoreMesh(core_axis_name="c", subcore_axis_name="s",
                                    num_cores=_NC, num_subcores=_NS),
        compiler_params=pltpu.CompilerParams(needs_layout_passes=False),
        scratch_types=[
            pltpu.VMEM((_APW * _B,), jnp.float32),
            pltpu.VMEM((_APW * _B,), jnp.float32),
            pltpu.VMEM((_L,), jnp.float32),
            pltpu.VMEM((_B + _L,), jnp.float32),
        ],
    )
    partials = sc(ap.reshape(_NW, _APW * _B), an.reshape(_NW, _APW * _B))
    return jnp.sum(partials)


# speedup vs baseline: 1.0655x; 1.0484x over previous
import functools

import jax
import jax.numpy as jnp
from jax import lax
from jax.experimental import pallas as pl
from jax.experimental.pallas import tpu as pltpu
from jax.experimental.pallas import tpu_sc as plsc

_B = 128
_MARGIN = 1.0
_EPS = 1e-8
_NC, _NS, _L = 2, 16, 16
_NW = _NC * _NS
_APW = _B // _NW
_NV = _B // _L


def _tc_body(embs_ref, lab_col_ref, lab_row_ref, pc_ref):
    e = embs_ref[...]
    g = lax.dot_general(e, e, (((1,), (1,)), ((), ())),
                        preferred_element_type=jnp.float32)
    n2c = jnp.sum(e * e, axis=1, keepdims=True)
    riota = lax.broadcasted_iota(jnp.int32, (_B, _B), 0)
    ciota = lax.broadcasted_iota(jnp.int32, (_B, _B), 1)
    diag_n2 = jnp.where(riota == ciota, jnp.broadcast_to(n2c, (_B, _B)), 0.0)
    n2r = lax.dot_general(jnp.ones((_B, _B), jnp.float32), diag_n2,
                          (((1,), (0,)), ((), ())),
                          preferred_element_type=jnp.float32)
    denom = jnp.maximum(jnp.sqrt(jnp.broadcast_to(n2c, (_B, _B)) * n2r), _EPS)
    s = g / denom

    same = jnp.broadcast_to(lab_col_ref[...], (_B, _B)) == \
        jnp.broadcast_to(lab_row_ref[...], (_B, _B))
    gt = ciota > riota
    pc_ref[...] = jnp.concatenate(
        [jnp.where(gt & same, s, 3.0),
         jnp.where(gt & (~same), s + _MARGIN, -3.0)], axis=1)


def _sc_body(pc_hbm, out_hbm, pc_v, pv, posb):
    wid = lax.axis_index("s") * _NC + lax.axis_index("c")
    pltpu.sync_copy(pc_hbm.at[wid], pc_v)
    def abody(a, accs):
        nn = [pc_v[pl.ds(a * 2 * _B + _B + v * _L, _L)] for v in range(_NV)]
        npos = jnp.int32(0)
        for v in range(_NV):
            w16 = pc_v[pl.ds(a * 2 * _B + v * _L, _L)]
            m = w16 < 2.0
            rank = plsc.cumsum(m.astype(jnp.int32)) - 1
            plsc.store_scatter(posb, [rank + npos], w16, mask=m)
            npos = npos + lax.reduce_max(
                plsc.all_reduce_population_count(m), (0,))

        def jbody(j, accs, nn=nn):
            p = plsc.load_gather(posb, [jnp.full((_L,), 0, jnp.int32) + j])
            return tuple(accs[v] + jnp.maximum(nn[v] - p, 0.0)
                         for v in range(_NV))

        return lax.fori_loop(0, npos, jbody, accs)

    accs = lax.fori_loop(0, _APW, abody,
                         (jnp.zeros((_L,), jnp.float32),) * _NV)
    tot = accs[0]
    for v in range(1, _NV):
        tot = tot + accs[v]
    pv[...] = tot
    pltpu.sync_copy(pv, out_hbm.at[wid])


def kernel(embs, indices):
    lab = indices.astype(jnp.int32)
    pc = pl.pallas_call(
        _tc_body,
        out_shape=jax.ShapeDtypeStruct((_B, 2 * _B), jnp.float32),
    )(embs, lab.reshape(_B, 1), lab.reshape(1, _B))

    sc = pl.kernel(
        _sc_body,
        out_type=jax.ShapeDtypeStruct((_NW, _L), jnp.float32),
        mesh=plsc.VectorSubcoreMesh(core_axis_name="c", subcore_axis_name="s",
                                    num_cores=_NC, num_subcores=_NS),
        compiler_params=pltpu.CompilerParams(needs_layout_passes=False),
        scratch_types=[
            pltpu.VMEM((_APW * 2 * _B,), jnp.float32),
            pltpu.VMEM((_L,), jnp.float32),
            pltpu.VMEM((_B + _L,), jnp.float32),
        ],
    )
    partials = sc(pc.reshape(_NW, _APW * 2 * _B))
    return jnp.sum(partials)


# speedup vs baseline: 1.0679x; 1.0023x over previous
import jax
import jax.numpy as jnp
from jax import lax
from jax.experimental import pallas as pl
from jax.experimental.pallas import tpu as pltpu
from jax.experimental.pallas import tpu_sc as plsc

_B = 128
_MARGIN = 1.0
_EPS = 1e-8
_NC, _NS, _L = 2, 16, 16
_NW = _NC * _NS
_APW = _B // _NW
_NV = _B // _L


def _tc_body(embs_ref, lab_col_ref, lab_row_ref, pc_ref):
    e = embs_ref[...]
    g = lax.dot_general(e, e, (((1,), (1,)), ((), ())),
                        preferred_element_type=jnp.float32)
    n2c = jnp.sum(e * e, axis=1, keepdims=True)
    riota = lax.broadcasted_iota(jnp.int32, (_B, _B), 0)
    ciota = lax.broadcasted_iota(jnp.int32, (_B, _B), 1)
    diag_n2 = jnp.where(riota == ciota, jnp.broadcast_to(n2c, (_B, _B)), 0.0)
    n2r = lax.dot_general(jnp.ones((_B, _B), jnp.float32), diag_n2,
                          (((1,), (0,)), ((), ())),
                          preferred_element_type=jnp.float32)
    denom = jnp.maximum(jnp.sqrt(jnp.broadcast_to(n2c, (_B, _B)) * n2r), _EPS)
    s = g / denom

    same = jnp.broadcast_to(lab_col_ref[...], (_B, _B)) == \
        jnp.broadcast_to(lab_row_ref[...], (_B, _B))
    gt = ciota > riota
    pc_ref[...] = jnp.concatenate(
        [jnp.where(gt & same, s, 3.0),
         jnp.where(gt & (~same), s + _MARGIN, -3.0)], axis=1)


def _sc_body(pc_hbm, out_hbm, pc_v, pv, posb):
    wid = lax.axis_index("s") * _NC + lax.axis_index("c")
    pltpu.sync_copy(pc_hbm.at[wid], pc_v)
    def abody(a, accs):
        nn = [pc_v[pl.ds(a * 2 * _B + _B + v * _L, _L)] for v in range(_NV)]
        npos = jnp.int32(0)
        for v in range(_NV):
            w16 = pc_v[pl.ds(a * 2 * _B + v * _L, _L)]
            m = w16 < 2.0
            rank = plsc.cumsum(m.astype(jnp.int32)) - 1
            plsc.store_scatter(posb, [rank + npos], w16, mask=m)
            npos = npos + lax.reduce_max(
                plsc.all_reduce_population_count(m), (0,))

        def jbody(j, accs, nn=nn):
            p = plsc.load_gather(posb, [jnp.full((_L,), 0, jnp.int32) + j])
            return tuple(accs[v] + jnp.maximum(nn[v] - p, 0.0)
                         for v in range(_NV))

        return lax.fori_loop(0, npos, jbody, accs)

    accs = lax.fori_loop(0, _APW, abody,
                         (jnp.zeros((_L,), jnp.float32),) * _NV)
    tot = accs[0]
    for v in range(1, _NV):
        tot = tot + accs[v]
    pv[...] = tot
    pltpu.sync_copy(pv, out_hbm.at[wid])


def kernel(embs, indices):
    lab = indices.astype(jnp.int32)
    pc = pl.pallas_call(
        _tc_body,
        out_shape=jax.ShapeDtypeStruct((_B, 2 * _B), jnp.float32),
    )(embs, lab.reshape(_B, 1), lab.reshape(1, _B))

    sc = pl.kernel(
        _sc_body,
        out_type=jax.ShapeDtypeStruct((_NW, _L), jnp.float32),
        mesh=plsc.VectorSubcoreMesh(core_axis_name="c", subcore_axis_name="s",
                                    num_cores=_NC, num_subcores=_NS),
        compiler_params=pltpu.CompilerParams(needs_layout_passes=False),
        scratch_types=[
            pltpu.VMEM((_APW * 2 * _B,), jnp.float32),
            pltpu.VMEM((_L,), jnp.float32),
            pltpu.VMEM((_B + _L,), jnp.float32),
        ],
    )
    partials = sc(pc.reshape(_NW, _APW * 2 * _B))
    return jnp.sum(partials)
